# baseline (speedup 1.0000x reference)
import functools
import jax
import jax.numpy as jnp
from jax import lax
from jax.experimental import pallas as pl
from jax.experimental.pallas import tpu as pltpu
from jax.experimental.pallas import tpu_sc as plsc

_BLK = 2048
_GW = 128


def _sc_gather(table, idx):
    info = plsc.get_sparse_core_info()
    nw = info.num_cores * info.num_subcores
    b = idx.shape[0]
    bpw = b // nw

    mesh = plsc.VectorSubcoreMesh(core_axis_name="c", subcore_axis_name="s")

    @functools.partial(
        pl.kernel,
        mesh=mesh,
        out_type=jax.ShapeDtypeStruct((b, _GW), jnp.float32),
        scratch_types=[
            pltpu.VMEM((bpw,), jnp.int32),
            pltpu.VMEM((bpw, _GW), jnp.float32),
            pltpu.SemaphoreType.DMA,
        ],
    )
    def k(table_hbm, idx_hbm, out_hbm, idx_v, rows_v, sem):
        wid = lax.axis_index("s") * info.num_cores + lax.axis_index("c")
        base = wid * bpw
        pltpu.sync_copy(idx_hbm.at[pl.ds(base, bpw)], idx_v)
        pltpu.async_copy(table_hbm.at[idx_v], rows_v, sem).wait()
        pltpu.sync_copy(rows_v, out_hbm.at[pl.ds(base, bpw)])

    return k(table, idx)


def _tc_body(x_ref, g_ref, w1_ref, b1_ref, w2_ref, b2_ref, wa_ref, wb_ref,
             b3_ref, w4_ref, out_ref):
    f = jnp.maximum(x_ref[...] @ w1_ref[...] + b1_ref[...], 0.0)
    f = jnp.maximum(f @ w2_ref[...] + b2_ref[...], 0.0)
    g = g_ref[...]
    h = jnp.maximum(f @ wa_ref[...] + g @ wb_ref[...] + b3_ref[...], 0.0)
    w = h * w4_ref[...]
    acc = g[:, 19:20]
    for i in range(3):
        pi = jnp.sum(w[:, i * 64:(i + 1) * 64], axis=1, keepdims=True)
        acc = acc + g[:, 16 + i:17 + i] * pi
    out_ref[...] = acc


def _tc_call(x, g, w1, b1r, w2, b2r, wa, wb, b3r, w4r):
    bsz = x.shape[0]
    full = lambda i: (0, 0)
    return pl.pallas_call(
        _tc_body,
        grid=(bsz // _BLK,),
        in_specs=[
            pl.BlockSpec((_BLK, 128), lambda i: (i, 0)),
            pl.BlockSpec((_BLK, _GW), lambda i: (i, 0)),
            pl.BlockSpec((128, 64), full),
            pl.BlockSpec((1, 64), full),
            pl.BlockSpec((64, 32), full),
            pl.BlockSpec((1, 32), full),
            pl.BlockSpec((32, 192), full),
            pl.BlockSpec((_GW, 192), full),
            pl.BlockSpec((1, 192), full),
            pl.BlockSpec((1, 192), full),
        ],
        out_specs=pl.BlockSpec((_BLK, 1), lambda i: (i, 0)),
        out_shape=jax.ShapeDtypeStruct((bsz, 1), jnp.float32),
        compiler_params=pltpu.CompilerParams(
            dimension_semantics=("arbitrary",)),
    )(x, g, w1, b1r, w2, b2r, wa, wb, b3r, w4r)


def kernel(x, regime_ids, W1, b1, W2, b2, emb, W3, b3, W4, b4):
    r, c, hh = W3.shape
    e = emb.shape[1]
    f2 = c - e

    idx = regime_ids.astype(jnp.int32)

    table = jnp.zeros((r, _GW), jnp.float32)
    table = table.at[:, :e].set(emb)
    table = table.at[:, e:e + r].set(jnp.eye(r, dtype=jnp.float32))
    table = table.at[:, e + r].set(b4[:, 0])

    g = _sc_gather(table, idx)

    w3cat = jnp.transpose(W3, (1, 0, 2)).reshape(c, r * hh)
    wa = w3cat[:f2]
    wb = jnp.zeros((_GW, r * hh), jnp.float32).at[:e].set(w3cat[f2:])
    b3r = b3.reshape(1, r * hh)
    w4r = W4.reshape(1, r * hh)
    b1r = b1.reshape(1, -1)
    b2r = b2.reshape(1, -1)

    return _tc_call(x, g, W1, b1r, W2, b2r, wa, wb, b3r, w4r)


# speedup vs baseline: 5.0689x; 5.0689x over previous
import functools
import jax
import jax.numpy as jnp
from jax import lax
from jax.experimental import pallas as pl
from jax.experimental.pallas import tpu as pltpu
from jax.experimental.pallas import tpu_sc as plsc

_BLK = 2048
_NE = 8
_L = 16


def _sc_select(p, idx):
    info = plsc.get_sparse_core_info()
    nw = info.num_cores * info.num_subcores
    b = idx.shape[0]
    bpw = b // nw

    mesh = plsc.VectorSubcoreMesh(core_axis_name="c", subcore_axis_name="s")

    @functools.partial(
        pl.kernel,
        mesh=mesh,
        out_type=jax.ShapeDtypeStruct((b,), jnp.float32),
        scratch_types=[
            pltpu.VMEM((bpw * _NE,), jnp.float32),
            pltpu.VMEM((bpw,), jnp.int32),
            pltpu.VMEM((bpw,), jnp.float32),
        ],
        compiler_params=pltpu.CompilerParams(needs_layout_passes=False),
    )
    def k(p_hbm, idx_hbm, out_hbm, p_v, idx_v, out_v):
        wid = lax.axis_index("s") * info.num_cores + lax.axis_index("c")
        base = wid * bpw
        pltpu.sync_copy(p_hbm.at[pl.ds(base * _NE, bpw * _NE)], p_v)
        pltpu.sync_copy(idx_hbm.at[pl.ds(base, bpw)], idx_v)
        for j in range(bpw // _L):
            iv = idx_v[pl.ds(j * _L, _L)]
            flat = (j * _L + lax.iota(jnp.int32, _L)) * _NE + iv
            out_v[pl.ds(j * _L, _L)] = plsc.load_gather(p_v, [flat])
        pltpu.sync_copy(out_v, out_hbm.at[pl.ds(base, bpw)])

    return k(p, idx)


def _tc_body(x_ref, reg_ref, w1_ref, b1_ref, w2_ref, b2_ref, wa_ref,
             embp_ref, wbe_ref, b3_ref, w4_ref, b4_ref, out_ref):
    ids = reg_ref[...]
    oh = (ids == lax.broadcasted_iota(jnp.int32, (ids.shape[0], _NE), 1))
    oh = oh.astype(jnp.float32)
    f = jnp.maximum(x_ref[...] @ w1_ref[...] + b1_ref[...], 0.0)
    f = jnp.maximum(f @ w2_ref[...] + b2_ref[...], 0.0)
    t = embp_ref[...] @ wbe_ref[...]
    h = jnp.maximum(f @ wa_ref[...] + oh @ t + b3_ref[...], 0.0)
    out_ref[...] = h @ w4_ref[...] + b4_ref[...]


def _tc_call(x, reg2d, w1, b1r, w2, b2r, wa, embp, wbe, b3r, w4blk, b4r):
    bsz = x.shape[0]
    full = lambda i: (0, 0)
    return pl.pallas_call(
        _tc_body,
        grid=(bsz // _BLK,),
        in_specs=[
            pl.BlockSpec((_BLK, 128), lambda i: (i, 0)),
            pl.BlockSpec((_BLK, 1), lambda i: (i, 0)),
            pl.BlockSpec((128, 64), full),
            pl.BlockSpec((1, 64), full),
            pl.BlockSpec((64, 32), full),
            pl.BlockSpec((1, 32), full),
            pl.BlockSpec((32, 192), full),
            pl.BlockSpec((8, 16), full),
            pl.BlockSpec((16, 192), full),
            pl.BlockSpec((1, 192), full),
            pl.BlockSpec((192, _NE), full),
            pl.BlockSpec((1, _NE), full),
        ],
        out_specs=pl.BlockSpec((_BLK, _NE), lambda i: (i, 0)),
        out_shape=jax.ShapeDtypeStruct((bsz, _NE), jnp.float32),
        compiler_params=pltpu.CompilerParams(
            dimension_semantics=("arbitrary",)),
    )(x, reg2d, w1, b1r, w2, b2r, wa, embp, wbe, b3r, w4blk, b4r)


def kernel(x, regime_ids, W1, b1, W2, b2, emb, W3, b3, W4, b4):
    r, c, hh = W3.shape
    e = emb.shape[1]
    f2 = c - e
    rh = r * hh

    idx = regime_ids.astype(jnp.int32)

    w3cat = jnp.transpose(W3, (1, 0, 2)).reshape(c, rh)
    wa = w3cat[:f2]
    wbe = w3cat[f2:]
    embp = jnp.zeros((_NE, e), jnp.float32).at[:r].set(emb)
    b3r = b3.reshape(1, rh)
    b1r = b1.reshape(1, -1)
    b2r = b2.reshape(1, -1)
    w4blk = jnp.zeros((rh, _NE), jnp.float32)
    for i in range(r):
        w4blk = w4blk.at[i * hh:(i + 1) * hh, i].set(W4[i, :, 0])
    b4r = jnp.zeros((1, _NE), jnp.float32).at[0, :r].set(b4[:, 0])

    p = _tc_call(x, idx.reshape(-1, 1), W1, b1r, W2, b2r, wa, embp, wbe,
                 b3r, w4blk, b4r)
    return _sc_select(p.reshape(-1), idx).reshape(-1, 1)


# speedup vs baseline: 6.8133x; 1.3441x over previous
import functools
import jax
import jax.numpy as jnp
from jax import lax
from jax.experimental import pallas as pl
from jax.experimental.pallas import tpu as pltpu
from jax.experimental.pallas import tpu_sc as plsc

_BLK = 2048
_NE = 8
_L = 16


def _sc_select(p, idx):
    info = plsc.get_sparse_core_info()
    nw = info.num_cores * info.num_subcores
    b = idx.shape[0]
    bpw = b // nw

    mesh = plsc.VectorSubcoreMesh(core_axis_name="c", subcore_axis_name="s")

    @functools.partial(
        pl.kernel,
        mesh=mesh,
        out_type=jax.ShapeDtypeStruct((b,), jnp.float32),
        scratch_types=[
            pltpu.VMEM((bpw * _NE,), jnp.float32),
            pltpu.VMEM((bpw,), jnp.int32),
            pltpu.VMEM((bpw,), jnp.float32),
        ],
        compiler_params=pltpu.CompilerParams(needs_layout_passes=False),
    )
    def k(p_hbm, idx_hbm, out_hbm, p_v, idx_v, out_v):
        wid = lax.axis_index("s") * info.num_cores + lax.axis_index("c")
        base = wid * bpw
        pltpu.sync_copy(p_hbm.at[pl.ds(base * _NE, bpw * _NE)], p_v)
        pltpu.sync_copy(idx_hbm.at[pl.ds(base, bpw)], idx_v)
        for j in range(bpw // _L):
            iv = idx_v[pl.ds(j * _L, _L)]
            flat = (j * _L + lax.iota(jnp.int32, _L)) * _NE + iv
            out_v[pl.ds(j * _L, _L)] = plsc.load_gather(p_v, [flat])
        pltpu.sync_copy(out_v, out_hbm.at[pl.ds(base, bpw)])

    return k(p, idx)


def _tc_body(x_ref, w1_ref, b1_ref, w2_ref, b2_ref, w3_ref, emb_ref,
             b3_ref, w4_ref, b4_ref, out_ref):
    f = jnp.maximum(x_ref[...] @ w1_ref[...] + b1_ref[...], 0.0)
    f = jnp.maximum(f @ w2_ref[...] + b2_ref[...], 0.0)
    cols = []
    nb = x_ref.shape[0]
    for i in range(3):
        t = emb_ref[i:i + 1, :] @ w3_ref[i, 32:, :] + b3_ref[i:i + 1, :]
        h = jnp.maximum(f @ w3_ref[i, :32, :] + t, 0.0)
        cols.append(h @ w4_ref[i] + b4_ref[i:i + 1, :])
    cols.append(jnp.zeros((nb, _NE - 3), jnp.float32))
    out_ref[...] = jnp.concatenate(cols, axis=1)


def _tc_call(x, w1, b1r, w2, b2r, w3, emb, b3, w4, b4):
    bsz = x.shape[0]
    full = lambda i: (0, 0)
    full3 = lambda i: (0, 0, 0)
    return pl.pallas_call(
        _tc_body,
        grid=(bsz // _BLK,),
        in_specs=[
            pl.BlockSpec((_BLK, 128), lambda i: (i, 0)),
            pl.BlockSpec((128, 64), full),
            pl.BlockSpec((1, 64), full),
            pl.BlockSpec((64, 32), full),
            pl.BlockSpec((1, 32), full),
            pl.BlockSpec((3, 48, 64), full3),
            pl.BlockSpec((3, 16), full),
            pl.BlockSpec((3, 64), full),
            pl.BlockSpec((3, 64, 1), full3),
            pl.BlockSpec((3, 1), full),
        ],
        out_specs=pl.BlockSpec((_BLK, _NE), lambda i: (i, 0)),
        out_shape=jax.ShapeDtypeStruct((bsz, _NE), jnp.float32),
        compiler_params=pltpu.CompilerParams(
            dimension_semantics=("arbitrary",)),
    )(x, w1, b1r, w2, b2r, w3, emb, b3, w4, b4)


def kernel(x, regime_ids, W1, b1, W2, b2, emb, W3, b3, W4, b4):
    idx = regime_ids.astype(jnp.int32)
    p = _tc_call(x, W1, b1.reshape(1, -1), W2, b2.reshape(1, -1),
                 W3, emb, b3, W4, b4)
    return _sc_select(p.reshape(-1), idx).reshape(-1, 1)
